# baseline (speedup 1.0000x reference)
import functools
import jax
import jax.numpy as jnp
from jax import lax
from jax.experimental import pallas as pl
from jax.experimental.pallas import tpu as pltpu
from jax.experimental.pallas import tpu_sc as plsc

_NC = 2
_NS = 16


def _magic_div(ncol, pmax):
    for shift in range(16, 31):
        mul = (1 << shift) // ncol + 1
        if all((p * mul) >> shift == p // ncol for p in range(pmax)):
            return mul, shift
    raise ValueError(f"no exact multiply-shift division for {ncol=}, {pmax=}")


def _gather_kernel_body(rows_per_w, b_per_w, ncol, div_mul, div_shift,
                        x_hbm, table_hbm, out_hbm,
                        stage_v, idx_v, rows_v, sem):
    wid = lax.axis_index("s") * _NC + lax.axis_index("c")
    pltpu.sync_copy(x_hbm.at[pl.ds(wid * rows_per_w, rows_per_w), :], stage_v)
    lane = lax.iota(jnp.int32, 16)

    def linearize(k, carry):
        p = k * 16 + lane
        r = lax.shift_right_logical(p * div_mul, div_shift)
        c = p - r * ncol
        idx_v[pl.ds(k * 16, 16)] = plsc.load_gather(stage_v, [r, c])
        return carry

    lax.fori_loop(0, b_per_w // 16, linearize, 0)
    pltpu.async_copy(table_hbm.at[idx_v], rows_v, sem).wait()
    pltpu.sync_copy(rows_v, out_hbm.at[pl.ds(wid * b_per_w, b_per_w)])


def kernel(x, emb_weight):
    B = x.shape[0] * x.shape[1]
    nw = _NC * _NS
    b_per_w = B // nw
    assert B % nw == 0 and b_per_w % 8 == 0

    idx = x.astype(jnp.int32)

    mesh = plsc.VectorSubcoreMesh(core_axis_name="c", subcore_axis_name="s")
    rows_per_w = x.shape[0] // nw
    ncol = x.shape[1]
    div_mul, div_shift = _magic_div(ncol, b_per_w)
    gather = pl.kernel(
        functools.partial(_gather_kernel_body, rows_per_w, b_per_w, ncol,
                          div_mul, div_shift),
        mesh=mesh,
        out_type=jax.ShapeDtypeStruct((B,), jnp.float32),
        scratch_types=[
            pltpu.VMEM((rows_per_w, ncol), jnp.int32),
            pltpu.VMEM((b_per_w,), jnp.int32),
            pltpu.VMEM((b_per_w,), jnp.float32),
            pltpu.SemaphoreType.DMA,
        ],
        compiler_params=pltpu.CompilerParams(
            use_tc_tiling_on_sc=False, needs_layout_passes=False
        ),
    )
    out = gather(idx, emb_weight.reshape(-1))
    return out.reshape(-1, emb_weight.shape[1])


# speedup vs baseline: 1.1229x; 1.1229x over previous
import functools
import jax
import jax.numpy as jnp
from jax import lax
from jax.experimental import pallas as pl
from jax.experimental.pallas import tpu as pltpu
from jax.experimental.pallas import tpu_sc as plsc

_NC = 2
_NS = 16


def _magic_div(ncol, pmax):
    for shift in range(16, 31):
        mul = (1 << shift) // ncol + 1
        if all((p * mul) >> shift == p // ncol for p in range(pmax)):
            return mul, shift
    raise ValueError(f"no exact multiply-shift division for {ncol=}, {pmax=}")


def _gather_kernel_body(rows_per_w, b_per_w, ncol, div_mul, div_shift,
                        x_hbm, table_hbm, out_hbm,
                        stage_v, idx_v, rows_v, sem):
    wid = lax.axis_index("s") * _NC + lax.axis_index("c")
    pltpu.sync_copy(x_hbm.at[pl.ds(wid * rows_per_w, rows_per_w), :], stage_v)
    lane = lax.iota(jnp.int32, 16)

    def linearize(k, carry):
        p = k * 16 + lane
        r = lax.shift_right_logical(p * div_mul, div_shift)
        c = p - r * ncol
        idx_v[pl.ds(k * 16, 16)] = plsc.load_gather(stage_v, [r, c])
        return carry

    lax.fori_loop(0, b_per_w // 16, linearize, 0)
    pltpu.async_copy(table_hbm.at[idx_v], rows_v, sem).wait()
    pltpu.sync_copy(rows_v, out_hbm.at[pl.ds(wid * b_per_w, b_per_w)])


def kernel(x, emb_weight):
    B = x.shape[0] * x.shape[1]
    nw = _NC * _NS
    b_per_w = B // nw
    assert B % nw == 0 and b_per_w % 8 == 0

    idx = x.astype(jnp.int32)

    mesh = plsc.VectorSubcoreMesh(core_axis_name="c", subcore_axis_name="s")
    rows_per_w = x.shape[0] // nw
    ncol = x.shape[1]
    div_mul, div_shift = _magic_div(ncol, b_per_w)
    gather = pl.kernel(
        functools.partial(_gather_kernel_body, rows_per_w, b_per_w, ncol,
                          div_mul, div_shift),
        mesh=mesh,
        out_type=jax.ShapeDtypeStruct((B,), jnp.float32),
        scratch_types=[
            pltpu.VMEM((rows_per_w, ncol), jnp.int32),
            pltpu.VMEM((b_per_w,), jnp.int32),
            pltpu.VMEM((b_per_w,), jnp.float32),
            pltpu.SemaphoreType.DMA,
        ],
        compiler_params=pltpu.CompilerParams(needs_layout_passes=False),
    )
    out = gather(idx, emb_weight.reshape(-1))
    return out.reshape(-1, emb_weight.shape[1])


# speedup vs baseline: 1.1230x; 1.0002x over previous
import functools
import jax
import jax.numpy as jnp
from jax import lax
from jax.experimental import pallas as pl
from jax.experimental.pallas import tpu as pltpu
from jax.experimental.pallas import tpu_sc as plsc

_NC = 2
_NS = 16


def _magic_div(ncol, pmax):
    for shift in range(16, 31):
        mul = (1 << shift) // ncol + 1
        if all((p * mul) >> shift == p // ncol for p in range(pmax)):
            return mul, shift
    raise ValueError(f"no exact multiply-shift division for {ncol=}, {pmax=}")


def _gather_kernel_body(rows_per_w, b_per_w, ncol, div_mul, div_shift,
                        x_hbm, table_hbm, out_hbm,
                        stage_v, idx_v, rows_v, sem):
    wid = lax.axis_index("s") * _NC + lax.axis_index("c")
    pltpu.sync_copy(x_hbm.at[pl.ds(wid * rows_per_w, rows_per_w), :], stage_v)
    lane = lax.iota(jnp.int32, 16)

    def linearize(k, carry):
        p = k * 16 + lane
        r = lax.shift_right_logical(p * div_mul, div_shift)
        c = p - r * ncol
        idx_v[pl.ds(k * 16, 16)] = plsc.load_gather(stage_v, [r, c])
        return carry

    lax.fori_loop(0, b_per_w // 16, linearize, 0)
    pltpu.async_copy(table_hbm.at[idx_v], rows_v, sem).wait()
    pltpu.sync_copy(rows_v, out_hbm.at[pl.ds(wid * b_per_w, b_per_w)])


def kernel(x, emb_weight):
    B = x.shape[0] * x.shape[1]
    nw = _NC * _NS
    b_per_w = B // nw
    assert B % nw == 0 and b_per_w % 8 == 0

    idx = x.astype(jnp.int32)

    mesh = plsc.VectorSubcoreMesh(core_axis_name="c", subcore_axis_name="s")
    rows_per_w = x.shape[0] // nw
    ncol = x.shape[1]
    div_mul, div_shift = _magic_div(ncol, b_per_w)
    gather = pl.kernel(
        functools.partial(_gather_kernel_body, rows_per_w, b_per_w, ncol,
                          div_mul, div_shift),
        mesh=mesh,
        out_type=jax.ShapeDtypeStruct((B,), jnp.float32),
        scratch_types=[
            pltpu.VMEM((rows_per_w, ncol), jnp.int32),
            pltpu.VMEM((b_per_w,), jnp.int32),
            pltpu.VMEM((b_per_w,), jnp.float32),
            pltpu.SemaphoreType.DMA,
        ],
        compiler_params=pltpu.CompilerParams(needs_layout_passes=False),
    )
    out = gather(idx, emb_weight.T.reshape(-1))
    return out.reshape(-1, emb_weight.shape[1])


# speedup vs baseline: 1.1232x; 1.0002x over previous
import functools
import jax
import jax.numpy as jnp
from jax import lax
from jax.experimental import pallas as pl
from jax.experimental.pallas import tpu as pltpu
from jax.experimental.pallas import tpu_sc as plsc

_NC = 2
_NS = 16


def _magic_div(ncol, pmax):
    for shift in range(16, 31):
        mul = (1 << shift) // ncol + 1
        if all((p * mul) >> shift == p // ncol for p in range(pmax)):
            return mul, shift
    raise ValueError(f"no exact multiply-shift division for {ncol=}, {pmax=}")


def _gather_kernel_body(rows_per_w, b_per_w, ncol, div_mul, div_shift,
                        x_hbm, table_hbm, out_hbm,
                        stage_v, idx_v, rows_v, sem):
    wid = lax.axis_index("s") * _NC + lax.axis_index("c")
    pltpu.sync_copy(x_hbm.at[pl.ds(wid * rows_per_w, rows_per_w), :], stage_v)
    lane = lax.iota(jnp.int32, 16)

    def linearize(k, carry):
        p = k * 16 + lane
        r = lax.shift_right_logical(p * div_mul, div_shift)
        c = p - r * ncol
        idx_v[pl.ds(k * 16, 16)] = plsc.load_gather(stage_v, [r, c])
        return carry

    lax.fori_loop(0, b_per_w // 16, linearize, 0)
    pltpu.async_copy(table_hbm.at[idx_v], rows_v, sem).wait()
    pltpu.sync_copy(rows_v, out_hbm.at[pl.ds(wid * b_per_w, b_per_w)])


def kernel(x, emb_weight):
    B = x.shape[0] * x.shape[1]
    nw = _NC * _NS
    b_per_w = B // nw
    assert B % nw == 0 and b_per_w % 8 == 0

    idx = x.astype(jnp.int32)

    mesh = plsc.VectorSubcoreMesh(core_axis_name="c", subcore_axis_name="s")
    rows_per_w = x.shape[0] // nw
    ncol = x.shape[1]
    div_mul, div_shift = _magic_div(ncol, b_per_w)
    gather = pl.kernel(
        functools.partial(_gather_kernel_body, rows_per_w, b_per_w, ncol,
                          div_mul, div_shift),
        mesh=mesh,
        out_type=jax.ShapeDtypeStruct((B,), jnp.float32),
        scratch_types=[
            pltpu.VMEM((rows_per_w, ncol), jnp.int32),
            pltpu.VMEM((b_per_w,), jnp.int32),
            pltpu.VMEM((b_per_w,), jnp.float32),
            pltpu.SemaphoreType.DMA,
        ],
        compiler_params=pltpu.CompilerParams(needs_layout_passes=False),
    )
    out = gather(idx, emb_weight[:, 0])
    return out.reshape(-1, emb_weight.shape[1])


# speedup vs baseline: 1.1494x; 1.0233x over previous
import functools
import jax
import jax.numpy as jnp
from jax import lax
from jax.experimental import pallas as pl
from jax.experimental.pallas import tpu as pltpu
from jax.experimental.pallas import tpu_sc as plsc

_NC = 2
_NS = 16


def _magic_div(ncol, pmax):
    for shift in range(16, 31):
        mul = (1 << shift) // ncol + 1
        if all((p * mul) >> shift == p // ncol for p in range(pmax)):
            return mul, shift
    raise ValueError(f"no exact multiply-shift division for {ncol=}, {pmax=}")


def _gather_kernel_body(rows_per_w, b_per_w, ncol, div_mul, div_shift,
                        xt_hbm, table_hbm, out_hbm,
                        stage_v, idx_v, rows_v, sem):
    wid = lax.axis_index("s") * _NC + lax.axis_index("c")
    pltpu.sync_copy(xt_hbm.at[:, pl.ds(wid * rows_per_w, rows_per_w)], stage_v)
    lane = lax.iota(jnp.int32, 16)

    def linearize(k, carry):
        p = k * 16 + lane
        r = lax.shift_right_logical(p * div_mul, div_shift)
        c = p - r * ncol
        idx_v[pl.ds(k * 16, 16)] = plsc.load_gather(stage_v, [c, r])
        return carry

    lax.fori_loop(0, b_per_w // 16, linearize, 0)
    pltpu.async_copy(table_hbm.at[idx_v], rows_v, sem).wait()
    pltpu.sync_copy(rows_v, out_hbm.at[pl.ds(wid * b_per_w, b_per_w)])


def kernel(x, emb_weight):
    B = x.shape[0] * x.shape[1]
    nw = _NC * _NS
    b_per_w = B // nw
    assert B % nw == 0 and b_per_w % 8 == 0

    idx = x.astype(jnp.int32)

    mesh = plsc.VectorSubcoreMesh(core_axis_name="c", subcore_axis_name="s")
    rows_per_w = x.shape[0] // nw
    ncol = x.shape[1]
    div_mul, div_shift = _magic_div(ncol, b_per_w)
    gather = pl.kernel(
        functools.partial(_gather_kernel_body, rows_per_w, b_per_w, ncol,
                          div_mul, div_shift),
        mesh=mesh,
        out_type=jax.ShapeDtypeStruct((B,), jnp.float32),
        scratch_types=[
            pltpu.VMEM((ncol, rows_per_w), jnp.int32),
            pltpu.VMEM((b_per_w,), jnp.int32),
            pltpu.VMEM((b_per_w,), jnp.float32),
            pltpu.SemaphoreType.DMA,
        ],
        compiler_params=pltpu.CompilerParams(needs_layout_passes=False),
    )
    out = gather(idx.T, emb_weight.reshape(-1))
    return out.reshape(-1, emb_weight.shape[1])
